# baseline (speedup 1.0000x reference)
import functools

import jax
import jax.numpy as jnp
from jax import lax
from jax.experimental import pallas as pl
from jax.experimental.pallas import tpu as pltpu
from jax.experimental.pallas import tpu_sc as plsc

SPEAKER_DIM = 128
CHUNK = 128


def _sc_lookup(num_rows, b_per_w, table_rows):
    mesh = plsc.VectorSubcoreMesh(core_axis_name="c", subcore_axis_name="s")
    num_chunks = b_per_w // CHUNK

    @functools.partial(
        pl.kernel,
        mesh=mesh,
        out_type=jax.ShapeDtypeStruct((num_rows, SPEAKER_DIM), jnp.float32),
        scratch_types=[
            pltpu.VMEM((CHUNK,), jnp.int32),
            pltpu.VMEM((CHUNK, SPEAKER_DIM), jnp.float32),
            pltpu.SemaphoreType.DMA,
        ],
    )
    def k(labels_hbm, table_hbm, out_hbm, idx_v, rows_v, sem):
        nc = 2
        wid = lax.axis_index("s") * nc + lax.axis_index("c")
        wbase = wid * b_per_w

        def body(i, carry):
            base = wbase + i * CHUNK
            pltpu.sync_copy(labels_hbm.at[pl.ds(base, CHUNK)], idx_v)
            pltpu.async_copy(table_hbm.at[idx_v], rows_v, sem).wait()
            pltpu.sync_copy(rows_v, out_hbm.at[pl.ds(base, CHUNK)])
            return carry

        lax.fori_loop(0, num_chunks, body, 0)

    return k


def kernel(speaker_labels, table):
    n, m = speaker_labels.shape
    num_rows = n * m
    labels_flat = speaker_labels.reshape(num_rows).astype(jnp.int32)
    b_per_w = num_rows // 32
    out = _sc_lookup(num_rows, b_per_w, table.shape[0])(labels_flat, table)
    return out.reshape(n, m, SPEAKER_DIM)


# speedup vs baseline: 27.6306x; 27.6306x over previous
import functools

import jax
import jax.numpy as jnp
from jax import lax
from jax.experimental import pallas as pl
from jax.experimental.pallas import tpu as pltpu
from jax.experimental.pallas import tpu_sc as plsc

SPEAKER_DIM = 128
CHUNK = 128


def _sc_lookup(num_rows, b_per_w, table_rows):
    mesh = plsc.VectorSubcoreMesh(core_axis_name="c", subcore_axis_name="s")
    num_chunks = b_per_w // CHUNK

    @functools.partial(
        pl.kernel,
        mesh=mesh,
        out_type=jax.ShapeDtypeStruct((num_rows, SPEAKER_DIM), jnp.float32),
        scratch_types=[
            pltpu.VMEM((CHUNK,), jnp.int32),
            pltpu.VMEM((CHUNK, SPEAKER_DIM), jnp.float32),
            pltpu.VMEM_SHARED((table_rows, SPEAKER_DIM), jnp.float32),
            pltpu.SemaphoreType.DMA,
        ],
    )
    def k(labels_hbm, table_hbm, out_hbm, idx_v, rows_v, table_sh, sem):
        nc = 2
        sid = lax.axis_index("s")
        wid = sid * nc + lax.axis_index("c")
        wbase = wid * b_per_w

        @pl.when(sid == 0)
        def _():
            pltpu.sync_copy(table_hbm, table_sh)

        plsc.subcore_barrier()

        def body(i, carry):
            base = wbase + i * CHUNK
            pltpu.sync_copy(labels_hbm.at[pl.ds(base, CHUNK)], idx_v)
            pltpu.async_copy(table_sh.at[idx_v], rows_v, sem).wait()
            pltpu.sync_copy(rows_v, out_hbm.at[pl.ds(base, CHUNK)])
            return carry

        lax.fori_loop(0, num_chunks, body, 0)

    return k


def kernel(speaker_labels, table):
    n, m = speaker_labels.shape
    num_rows = n * m
    labels_flat = speaker_labels.reshape(num_rows).astype(jnp.int32)
    b_per_w = num_rows // 32
    out = _sc_lookup(num_rows, b_per_w, table.shape[0])(labels_flat, table)
    return out.reshape(n, m, SPEAKER_DIM)
